# baseline (speedup 1.0000x reference)
import functools

import jax
import jax.numpy as jnp
from jax.experimental import pallas as pl
from jax.experimental.pallas import tpu as pltpu


def _mm_kernel(x_ref, w_ref, b_ref, o_ref, xb_ref, acc_ref, *, nk, nj):
    k = pl.program_id(1)
    j = pl.program_id(2)

    @pl.when(j == 0)
    def _():
        xb_ref[...] = x_ref[...].astype(jnp.bfloat16)

    part = jnp.dot(xb_ref[...], w_ref[...], preferred_element_type=jnp.float32)

    @pl.when(k < nk - 1)
    def _():
        acc_ref[j] = part

    @pl.when(k == nk - 1)
    def _():
        o_ref[...] = jnp.tanh(acc_ref[j] + part + b_ref[...])


@jax.jit
def _linear_tanh_fused(x2, w_t, b2):
    n, kdim = x2.shape
    m = w_t.shape[1]
    tm, tn, tk = 1024, 1024, 2048
    tm = min(tm, n)
    tn = min(tn, m)
    tk = min(tk, kdim)
    ni, nj, nk = pl.cdiv(n, tm), pl.cdiv(m, tn), pl.cdiv(kdim, tk)

    wb = w_t.astype(jnp.bfloat16)
    body = functools.partial(_mm_kernel, nk=nk, nj=nj)

    return pl.pallas_call(
        body,
        out_shape=jax.ShapeDtypeStruct((n, m), jnp.float32),
        grid=(ni, nk, nj),
        in_specs=[
            pl.BlockSpec((tm, tk), lambda i, k, j: (i, k)),
            pl.BlockSpec((tk, tn), lambda i, k, j: (k, j)),
            pl.BlockSpec((1, tn), lambda i, k, j: (0, j)),
        ],
        out_specs=pl.BlockSpec(
            (tm, tn),
            lambda i, k, j: (i, jnp.where(k == nk - 1, j, 0)),
        ),
        scratch_shapes=[
            pltpu.VMEM((tm, tk), jnp.bfloat16),
            pltpu.VMEM((nj, tm, tn), jnp.float32),
        ],
        compiler_params=pltpu.CompilerParams(
            dimension_semantics=("parallel", "arbitrary", "arbitrary"),
            vmem_limit_bytes=64 * 1024 * 1024,
        ),
    )(x2, wb, b2)


def kernel(x, w_t, b2):
    in_ch = w_t.shape[0]
    x2 = x.reshape(-1, in_ch)
    return _linear_tanh_fused(x2, w_t, b2)


# speedup vs baseline: 1.1289x; 1.1289x over previous
import jax
import jax.numpy as jnp
from jax.experimental import pallas as pl
from jax.experimental.pallas import tpu as pltpu


_TM = 256


def _mm_kernel(x_ref, w_ref, b_ref, o_ref):
    xb = x_ref[...].astype(jnp.bfloat16)
    acc = jnp.dot(xb, w_ref[...], preferred_element_type=jnp.float32)
    o_ref[...] = jnp.tanh(acc + b_ref[...])


@jax.jit
def _linear_tanh_fused(x2, w_t, b2):
    n, k = x2.shape
    m = w_t.shape[1]
    tm = min(_TM, n)
    ni = pl.cdiv(n, tm)

    wb = w_t.astype(jnp.bfloat16)

    return pl.pallas_call(
        _mm_kernel,
        out_shape=jax.ShapeDtypeStruct((n, m), jnp.float32),
        grid=(ni,),
        in_specs=[
            pl.BlockSpec((tm, k), lambda i: (i, 0)),
            pl.BlockSpec((k, m), lambda i: (0, 0)),
            pl.BlockSpec((1, m), lambda i: (0, 0)),
        ],
        out_specs=pl.BlockSpec((tm, m), lambda i: (i, 0)),
        compiler_params=pltpu.CompilerParams(
            dimension_semantics=("parallel",),
            vmem_limit_bytes=64 * 1024 * 1024,
        ),
    )(x2, wb, b2)


def kernel(x, w_t, b2):
    in_ch = w_t.shape[0]
    x2 = x.reshape(-1, in_ch)
    return _linear_tanh_fused(x2, w_t, b2)
